# baseline (device time: 14834 ns/iter reference)
import jax
import jax.numpy as jnp
from jax import lax
from jax.experimental import pallas as pl
from jax.experimental.pallas import tpu as pltpu

N_DEV = 8


def kernel(x):
    m_rows, n_cols = x.shape

    def body(x_ref, out_ref, e_ref, gather_ref, send_sems, recv_sems):
        my_pos = lax.axis_index("i")

        barrier_sem = pltpu.get_barrier_semaphore()
        for k in range(1, N_DEV):
            pl.semaphore_signal(
                barrier_sem, inc=1,
                device_id=(my_pos ^ k,), device_id_type=pl.DeviceIdType.MESH,
            )

        xv = x_ref[:, :]
        m = jnp.max(xv, axis=1)
        e = jnp.exp(xv - m[:, None])
        s = jnp.sum(e, axis=1)
        gather_ref[0] = jnp.stack([m, s], axis=0)

        pl.semaphore_wait(barrier_sem, N_DEV - 1)

        sends = {}
        for k in range(1, N_DEV):
            rdma = pltpu.make_async_remote_copy(
                src_ref=gather_ref.at[0],
                dst_ref=gather_ref.at[k],
                send_sem=send_sems.at[k],
                recv_sem=recv_sems.at[k],
                device_id=(my_pos ^ k,),
                device_id_type=pl.DeviceIdType.MESH,
            )
            rdma.start()
            sends[k] = rdma

        e_ref[:, :] = e.astype(jnp.bfloat16)

        rm, rs = m, s
        for k in (1, 3, 4, 2, 5, 7, 6):
            sends[k].wait_recv()
            bm = gather_ref[k, 0, :]
            bs = gather_ref[k, 1, :]
            nm = jnp.maximum(rm, bm)
            rs = rs * jnp.exp(rm - nm) + bs * jnp.exp(bm - nm)
            rm = nm

        scale = jnp.exp(m - rm) / rs
        out_ref[:, :] = e_ref[:, :] * scale.astype(jnp.bfloat16)[:, None]

        for k in range(1, N_DEV):
            sends[k].wait_send()

    return pl.pallas_call(
        body,
        out_shape=jax.ShapeDtypeStruct((m_rows, n_cols), jnp.bfloat16),
        in_specs=[pl.BlockSpec(memory_space=pltpu.VMEM)],
        out_specs=pl.BlockSpec(memory_space=pltpu.VMEM),
        scratch_shapes=[
            pltpu.VMEM((m_rows, n_cols), jnp.bfloat16),
            pltpu.VMEM((N_DEV, 2, m_rows), jnp.float32),
            pltpu.SemaphoreType.DMA((N_DEV,)),
            pltpu.SemaphoreType.DMA((N_DEV,)),
        ],
        compiler_params=pltpu.CompilerParams(collective_id=0),
    )(x)


# device time: 9886 ns/iter; 1.5005x vs baseline; 1.5005x over previous
import jax
import jax.numpy as jnp
from jax import lax
from jax.experimental import pallas as pl
from jax.experimental.pallas import tpu as pltpu

N_DEV = 8


def kernel(x):
    m_rows, n_cols = x.shape

    def body(x_ref, out_ref, e_ref, gather_ref, send_sems, recv_sems):
        my_pos = lax.axis_index("i")

        barrier_sem = pltpu.get_barrier_semaphore()
        for k in range(1, N_DEV):
            pl.semaphore_signal(
                barrier_sem, inc=1,
                device_id=(my_pos ^ k,), device_id_type=pl.DeviceIdType.MESH,
            )

        xv = x_ref[:, :]
        m = jnp.max(xv, axis=1)
        e = jnp.exp(xv - m[:, None])
        s = jnp.sum(e, axis=1)
        gather_ref[0] = jnp.stack([m, s], axis=0)

        pl.semaphore_wait(barrier_sem, N_DEV - 1)

        SLOT = {1: 1, 3: 2, 4: 3, 2: 4, 5: 5, 7: 6, 6: 7}
        sends = {}
        for k in (1, 3, 4, 2, 5, 7, 6):
            rdma = pltpu.make_async_remote_copy(
                src_ref=gather_ref.at[0],
                dst_ref=gather_ref.at[SLOT[k]],
                send_sem=send_sems.at[SLOT[k]],
                recv_sem=recv_sems.at[SLOT[k]],
                device_id=(my_pos ^ k,),
                device_id_type=pl.DeviceIdType.MESH,
            )
            rdma.start()
            sends[k] = rdma

        e_ref[:, :] = e.astype(jnp.bfloat16)

        for k in (1, 3, 4, 2, 5, 7):
            sends[k].wait_recv()
        near_m = gather_ref[0:7, 0, :]
        near_s = gather_ref[0:7, 1, :]
        pmax = jnp.max(near_m, axis=0)
        psum = jnp.sum(near_s * jnp.exp(near_m - pmax[None, :]), axis=0)

        sends[6].wait_recv()
        cm = gather_ref[7, 0, :]
        cs = gather_ref[7, 1, :]
        gmax = jnp.maximum(pmax, cm)
        gsum = psum * jnp.exp(pmax - gmax) + cs * jnp.exp(cm - gmax)

        scale = jnp.exp(m - gmax) / gsum
        out_ref[:, :] = e_ref[:, :] * scale.astype(jnp.bfloat16)[:, None]

        for k in range(1, N_DEV):
            sends[k].wait_send()

    return pl.pallas_call(
        body,
        out_shape=jax.ShapeDtypeStruct((m_rows, n_cols), jnp.bfloat16),
        in_specs=[pl.BlockSpec(memory_space=pltpu.VMEM)],
        out_specs=pl.BlockSpec(memory_space=pltpu.VMEM),
        scratch_shapes=[
            pltpu.VMEM((m_rows, n_cols), jnp.bfloat16),
            pltpu.VMEM((N_DEV, 2, m_rows), jnp.float32),
            pltpu.SemaphoreType.DMA((N_DEV,)),
            pltpu.SemaphoreType.DMA((N_DEV,)),
        ],
        compiler_params=pltpu.CompilerParams(collective_id=0),
    )(x)
